# Initial kernel scaffold; baseline (speedup 1.0000x reference)
import jax
import jax.numpy as jnp
from jax.experimental import pallas as pl


def kernel(enrollment, start_date, study_type, time, edge_index, num_W, num_b, ln_num_g, ln_num_b, emb, lstm_Wih, lstm_Whh, lstm_bih, lstm_bhh, fa_Wi, fa_bi, fa_Wo, fa_bo, conv_W, conv_b, ta_Wi, ta_bi, ta_Wo, ta_bo, te_W, te_b, sage1_Wl, sage1_bl, sage1_Wr, sage2_Wl, sage2_bl, sage2_Wr, ln1_g, ln1_b, ln2_g, ln2_b, out_W, out_b):
    raise NotImplementedError("write your pallas kernel here")


# speedup vs baseline: 17.4398x; 17.4398x over previous
import functools

import jax
import jax.numpy as jnp
from jax import lax
from jax.experimental import pallas as pl
from jax.experimental.pallas import tpu as pltpu
from jax.experimental.pallas import tpu_sc as plsc

_N = 10000
_E = 160000
_H = 128
_NH = 4
_DH = _H // _NH
_VOCAB = 1000
_BLK = 1000
_EC = 128
_EPT = _E // 16
_RPT = _N // 16
_HF = 96


def _mm(a, b):
    return jax.lax.dot_general(
        a, b, (((a.ndim - 1,), (0,)), ((), ())),
        precision=jax.lax.Precision.HIGHEST,
        preferred_element_type=jnp.float32)


def _ln(x, g, b):
    mu = jnp.mean(x, axis=-1, keepdims=True)
    var = jnp.mean((x - mu) ** 2, axis=-1, keepdims=True)
    return (x - mu) * jax.lax.rsqrt(var + 1e-5) * g + b


def _mha3(xq, xv, WqT, bq, WkT, bk, WvT, bv, WoT, bo, R, Ex):
    q = [_mm(t, WqT) + bq for t in xq]
    k = [_mm(t, WkT) + bk for t in xq]
    v = [_mm(t, WvT) + bv for t in xv]
    scale = 1.0 / (_DH ** 0.5)
    out = []
    for i in range(3):
        s = [_mm(q[i] * k[j], R) * scale for j in range(3)]
        m = jnp.maximum(jnp.maximum(s[0], s[1]), s[2])
        e = [jnp.exp(t - m) for t in s]
        den = e[0] + e[1] + e[2]
        o = sum(_mm(e[j] / den, Ex) * v[j] for j in range(3))
        out.append(_mm(o, WoT) + bo)
    return out


def _encode_body(feats, emb, numW, numb, lng, lnb, lstmW, lstmb,
                 faqT, fabq, fakT, fabk, favT, fabv, faoT, fabo,
                 cwT, cb, taqT, tabq, takT, tabk, tavT, tabv, taoT, tabo,
                 teW, teb, R, Ex, out_ref):
    f = feats[...]
    enr, sd = f[:, 0:1], f[:, 1:2]
    ts = [f[:, 2 + i:3 + i] for i in range(3)]
    stf = f[:, 5:6]

    e_num = jax.nn.relu(_ln(enr * numW[...] + numb[...], lng[...], lnb[...]))

    gates = sd * lstmW[...] + lstmb[...]
    gi, gg, go = gates[:, :_H], gates[:, 2 * _H:3 * _H], gates[:, 3 * _H:]
    c = jax.nn.sigmoid(gi) * jnp.tanh(gg)
    e_temp = jax.nn.sigmoid(go) * jnp.tanh(c)

    iota = lax.broadcasted_iota(
        jnp.int32, (f.shape[0], _VOCAB), 1).astype(jnp.float32)
    onehot = (stf == iota).astype(jnp.float32)
    e_cat = _mm(onehot, emb[...])

    x = _mha3([e_num, e_temp, e_cat], [e_num, e_temp, e_cat],
              faqT[...], fabq[...], fakT[...], fabk[...], favT[...],
              fabv[...], faoT[...], fabo[...], R[...], Ex[...])

    w0, w1, w2 = cwT[0], cwT[1], cwT[2]
    xc = [
        _mm(x[0], w1) + _mm(x[1], w2) + cb[...],
        _mm(x[0], w0) + _mm(x[1], w1) + _mm(x[2], w2) + cb[...],
        _mm(x[1], w0) + _mm(x[2], w1) + cb[...],
    ]
    qin = [xc[i] + ts[i] * teW[...] + teb[...] for i in range(3)]
    y = _mha3(qin, xc, taqT[...], tabq[...], takT[...], tabk[...],
              tavT[...], tabv[...], taoT[...], tabo[...], R[...], Ex[...])
    for i in range(3):
        out_ref[:, i * _H:(i + 1) * _H] = y[i]


def _sage_body(final, xin, s, cnt, WlT, bl, WrT, g, b, owT, ob, out_ref):
    inv = 1.0 / jnp.maximum(cnt[:, 0:1], 1.0)
    for i in range(3):
        sl = slice(i * _H, (i + 1) * _H)
        xi = xin[:, sl]
        h = _mm(s[:, sl] * inv, WlT[...]) + bl[...] + _mm(xi, WrT[...])
        h = jax.nn.relu(_ln(h, g[...], b[...])) + xi
        if final:
            out_ref[:, i, :] = _mm(h, owT[...]) + ob[...]
        else:
            out_ref[:, sl] = h


def _full(shape):
    return pl.BlockSpec(shape, lambda i: tuple(0 for _ in shape))


def _encode(feats, emb, *ws):
    grid = _N // _BLK
    in_specs = [pl.BlockSpec((_BLK, 8), lambda i: (i, 0)),
                _full((_VOCAB, _H))] + [_full(w.shape) for w in ws]
    return pl.pallas_call(
        _encode_body,
        grid=(grid,),
        in_specs=in_specs,
        out_specs=pl.BlockSpec((_BLK, 3 * _H), lambda i: (i, 0)),
        out_shape=jax.ShapeDtypeStruct((_N, 3 * _H), jnp.float32),
    )(feats, emb, *ws)


def _sage(final, xin, s, cnt, *ws):
    grid = _N // _BLK
    in_specs = [pl.BlockSpec((_BLK, 3 * _H), lambda i: (i, 0)),
                pl.BlockSpec((_BLK, 3 * _H), lambda i: (i, 0)),
                pl.BlockSpec((_BLK, 8), lambda i: (i, 0))]
    in_specs += [_full(w.shape) for w in ws]
    if final:
        out_spec = pl.BlockSpec((_BLK, 3, _H), lambda i: (i, 0, 0))
        out_shape = jax.ShapeDtypeStruct((_N, 3, _H), jnp.float32)
    else:
        out_spec = pl.BlockSpec((_BLK, 3 * _H), lambda i: (i, 0))
        out_shape = jax.ShapeDtypeStruct((_N, 3 * _H), jnp.float32)
    return pl.pallas_call(
        functools.partial(_sage_body, final),
        grid=(grid,),
        in_specs=in_specs,
        out_specs=out_spec,
        out_shape=out_shape,
    )(xin, s, cnt, *ws)


def _make_seg(compute_cnt):
    mesh = plsc.VectorSubcoreMesh(core_axis_name="c", subcore_axis_name="s")
    out_type = [jax.ShapeDtypeStruct((_N, _HF), jnp.float32),
                jax.ShapeDtypeStruct((_N, _HF), jnp.float32)]
    scratch = [
        pltpu.VMEM((_EC,), jnp.int32),
        pltpu.VMEM((_EC,), jnp.int32),
        pltpu.VMEM((_EC, _HF), jnp.float32),
        pltpu.VMEM((16,), jnp.int32),
        pltpu.VMEM((16,), jnp.int32),
        pltpu.VMEM((16, _HF), jnp.float32),
        pltpu.VMEM((_EC, 8), jnp.float32),
        pltpu.VMEM_SHARED((_N, _HF), jnp.float32),
        pltpu.VMEM_SHARED((_N, 8), jnp.float32),
        pltpu.SemaphoreType.DMA,
    ]
    if compute_cnt:
        out_type.append(jax.ShapeDtypeStruct((_N, 8), jnp.float32))

    def body(xa, xb, src, dst, z192, z8, ones, *refs):
        if compute_cnt:
            sa, sb, cnt_out = refs[0], refs[1], refs[2]
            rest = refs[3:]
        else:
            sa, sb = refs[0], refs[1]
            rest = refs[2:]
        (idx_s, idx_d, rows, idx_st, idx_dt, rows_t, ones_v,
         acc, cacc, sem) = rest
        cid = lax.axis_index("c")
        sid = lax.axis_index("s")

        def _rows(fn):
            @pl.when(sid < 15)
            def _():
                fn(pl.multiple_of(sid * 624, 8), 624)

            @pl.when(sid == 15)
            def _():
                fn(15 * 624, _N - 15 * 624)

        def _init(r0, nr):
            pltpu.sync_copy(z192.at[pl.ds(r0, nr)], acc.at[pl.ds(r0, nr)])
            if compute_cnt:
                pltpu.sync_copy(z8.at[pl.ds(r0, nr)], cacc.at[pl.ds(r0, nr)])

        _rows(_init)
        if compute_cnt:
            pltpu.sync_copy(ones, ones_v)
        plsc.subcore_barrier()

        nfull = _EPT // _EC
        base0 = sid * _EPT

        def chunk(i, _):
            base = base0 + i * _EC
            pltpu.sync_copy(src.at[pl.ds(base, _EC)], idx_s)
            pltpu.sync_copy(dst.at[pl.ds(base, _EC)], idx_d)

            @pl.when(cid == 0)
            def _():
                pltpu.async_copy(xa.at[idx_s], rows, sem).wait()

            @pl.when(cid == 1)
            def _():
                pltpu.async_copy(xb.at[idx_s], rows, sem).wait()

            pltpu.sync_copy(rows, acc.at[idx_d], add=True)
            if compute_cnt:
                @pl.when(cid == 0)
                def _():
                    pltpu.sync_copy(ones_v, cacc.at[idx_d], add=True)
            return 0

        lax.fori_loop(0, nfull, chunk, 0)

        tail = _EPT - nfull * _EC
        if tail:
            base = base0 + nfull * _EC
            pltpu.sync_copy(src.at[pl.ds(base, tail)], idx_st)
            pltpu.sync_copy(dst.at[pl.ds(base, tail)], idx_dt)

            @pl.when(cid == 0)
            def _():
                pltpu.async_copy(xa.at[idx_st], rows_t, sem).wait()

            @pl.when(cid == 1)
            def _():
                pltpu.async_copy(xb.at[idx_st], rows_t, sem).wait()

            pltpu.sync_copy(rows_t, acc.at[idx_dt], add=True)
            if compute_cnt:
                @pl.when(cid == 0)
                def _():
                    pltpu.sync_copy(ones_v.at[pl.ds(0, tail)],
                                    cacc.at[idx_dt], add=True)

        plsc.subcore_barrier()

        def _writeback(r0, nr):
            @pl.when(cid == 0)
            def _():
                pltpu.sync_copy(acc.at[pl.ds(r0, nr)], sa.at[pl.ds(r0, nr)])
                if compute_cnt:
                    pltpu.sync_copy(cacc.at[pl.ds(r0, nr)],
                                    cnt_out.at[pl.ds(r0, nr)])

            @pl.when(cid == 1)
            def _():
                pltpu.sync_copy(acc.at[pl.ds(r0, nr)], sb.at[pl.ds(r0, nr)])

        _rows(_writeback)

    return pl.kernel(
        body, out_type=tuple(out_type), mesh=mesh, scratch_types=scratch,
        compiler_params=pltpu.CompilerParams(use_tc_tiling_on_sc=False))


_make_seg = functools.lru_cache(maxsize=None)(_make_seg)


def _seg_cnt(*args):
    return _make_seg(True)(*args)


def _seg(*args):
    return _make_seg(False)(*args)


def kernel(enrollment, start_date, study_type, time, edge_index, num_W,
           num_b, ln_num_g, ln_num_b, emb, lstm_Wih, lstm_Whh, lstm_bih,
           lstm_bhh, fa_Wi, fa_bi, fa_Wo, fa_bo, conv_W, conv_b, ta_Wi,
           ta_bi, ta_Wo, ta_bo, te_W, te_b, sage1_Wl, sage1_bl, sage1_Wr,
           sage2_Wl, sage2_bl, sage2_Wr, ln1_g, ln1_b, ln2_g, ln2_b,
           out_W, out_b):
    f32 = jnp.float32
    feats = jnp.concatenate(
        [enrollment[:, None], start_date[:, None], time,
         study_type.astype(f32)[:, None], jnp.zeros((_N, 2), f32)], axis=1)

    r1 = lambda v: v.reshape(1, -1)
    eye = jnp.eye(_NH, dtype=f32)
    R = jnp.repeat(eye, _DH, axis=0)
    Ex = R.T
    enc_ws = (
        r1(num_W[:, 0]), r1(num_b), r1(ln_num_g), r1(ln_num_b),
        r1(lstm_Wih[:, 0]), r1(lstm_bih + lstm_bhh),
        fa_Wi[:_H].T, r1(fa_bi[:_H]), fa_Wi[_H:2 * _H].T,
        r1(fa_bi[_H:2 * _H]), fa_Wi[2 * _H:].T, r1(fa_bi[2 * _H:]),
        fa_Wo.T, r1(fa_bo),
        jnp.transpose(conv_W, (2, 1, 0)), r1(conv_b),
        ta_Wi[:_H].T, r1(ta_bi[:_H]), ta_Wi[_H:2 * _H].T,
        r1(ta_bi[_H:2 * _H]), ta_Wi[2 * _H:].T, r1(ta_bi[2 * _H:]),
        ta_Wo.T, r1(ta_bo),
        r1(te_W[:, 0]), r1(te_b), R, Ex,
    )
    x = _encode(feats, emb, *enc_ws)

    src = edge_index[0].astype(jnp.int32)
    dst = edge_index[1].astype(jnp.int32)
    z192 = jnp.zeros((_N, _HF), f32)
    z8 = jnp.zeros((_N, 8), f32)
    ones = jnp.ones((_EC, 8), f32)

    def seg_all(v, first):
        parts = [v[:, i * _HF:(i + 1) * _HF] for i in range(4)]
        if first:
            a0, b0, cnt = _seg_cnt(parts[0], parts[1], src, dst, z192, z8,
                                   ones)
        else:
            a0, b0 = _seg(parts[0], parts[1], src, dst, z192, z8, ones)
            cnt = None
        a1, b1 = _seg(parts[2], parts[3], src, dst, z192, z8, ones)
        return jnp.concatenate([a0, b0, a1, b1], axis=1), cnt

    s, cnt = seg_all(x, True)
    h = _sage(False, x, s, cnt, sage1_Wl.T, r1(sage1_bl), sage1_Wr.T,
              r1(ln1_g), r1(ln1_b), out_W.T, r1(out_b))

    s2, _ = seg_all(h, False)
    return _sage(True, h, s2, cnt, sage2_Wl.T, r1(sage2_bl), sage2_Wr.T,
                 r1(ln2_g), r1(ln2_b), out_W.T, r1(out_b))
